# Initial kernel scaffold; baseline (speedup 1.0000x reference)
import jax
import jax.numpy as jnp
from jax.experimental import pallas as pl


def kernel(cand_video_id, cand_author_id, hist_video_id, hist_author_id, video_table, author_table, W1, b1, a1, W2, b2, a2, W3, b3, Wh, bh):
    raise NotImplementedError("write your pallas kernel here")


# speedup vs baseline: 1.0509x; 1.0509x over previous
import functools

import jax
import jax.numpy as jnp
from jax import lax
from jax.experimental import pallas as pl
from jax.experimental.pallas import tpu as pltpu
from jax.experimental.pallas import tpu_sc as plsc

B = 4096
L = 200
VBUCK = 1000000
ABUCK = 500000
VDIM = 32
ADIM = 16

NC = 2
NS = 16
NW = NC * NS
CH = 128

NH = B * L
HPW = NH // NW
CPW = B // NW
NCHUNK = HPW // CH


def _hash_chunk(idx_ref, nbuck):
    for t in range(CH // 16):
        v = idx_ref[pl.ds(t * 16, 16)]
        h = v % (nbuck - 1) + 1
        idx_ref[pl.ds(t * 16, 16)] = jnp.where(v == 0, 0, h)


def _sc_gather_body(hvid, haid, cvid, caid, vtab, atab,
                    hv_out, ha_out, cv_out, ca_out,
                    idxv, idxa, rowsv, rowsa, semv, sema):
    wid = lax.axis_index("s") * NC + lax.axis_index("c")

    cbase = wid * CPW
    pltpu.sync_copy(cvid.at[pl.ds(cbase, CH)], idxv)
    pltpu.sync_copy(caid.at[pl.ds(cbase, CH)], idxa)
    _hash_chunk(idxv, VBUCK)
    _hash_chunk(idxa, ABUCK)
    cpv = pltpu.async_copy(vtab.at[idxv], rowsv, semv)
    cpa = pltpu.async_copy(atab.at[idxa], rowsa, sema)
    cpv.wait()
    cpa.wait()
    pltpu.sync_copy(rowsv, cv_out.at[pl.ds(cbase, CH)])
    pltpu.sync_copy(rowsa, ca_out.at[pl.ds(cbase, CH)])

    def body(j, _):
        base = wid * HPW + j * CH
        pltpu.sync_copy(hvid.at[pl.ds(base, CH)], idxv)
        pltpu.sync_copy(haid.at[pl.ds(base, CH)], idxa)
        _hash_chunk(idxv, VBUCK)
        _hash_chunk(idxa, ABUCK)
        gv = pltpu.async_copy(vtab.at[idxv], rowsv, semv)
        ga = pltpu.async_copy(atab.at[idxa], rowsa, sema)
        gv.wait()
        ga.wait()
        pltpu.sync_copy(rowsv, hv_out.at[pl.ds(base, CH)])
        pltpu.sync_copy(rowsa, ha_out.at[pl.ds(base, CH)])
        return 0

    lax.fori_loop(0, NCHUNK, body, 0)


def _sc_gather(hvid_flat, haid_flat, cvid, caid, vtab, atab):
    mesh = plsc.VectorSubcoreMesh(core_axis_name="c", subcore_axis_name="s")
    return pl.kernel(
        _sc_gather_body,
        out_type=[
            jax.ShapeDtypeStruct((NH, VDIM), jnp.float32),
            jax.ShapeDtypeStruct((NH, ADIM), jnp.float32),
            jax.ShapeDtypeStruct((B, VDIM), jnp.float32),
            jax.ShapeDtypeStruct((B, ADIM), jnp.float32),
        ],
        mesh=mesh,
        scratch_types=[
            pltpu.VMEM((CH,), jnp.int32),
            pltpu.VMEM((CH,), jnp.int32),
            pltpu.VMEM((CH, VDIM), jnp.float32),
            pltpu.VMEM((CH, ADIM), jnp.float32),
            pltpu.SemaphoreType.DMA,
            pltpu.SemaphoreType.DMA,
        ],
        compiler_params=pltpu.CompilerParams(use_tc_tiling_on_sc=False),
    )(hvid_flat, haid_flat, cvid, caid, vtab, atab)


BB = 32


def _tc_din_body(hv_ref, ha_ref, hvid_ref, cv_ref, ca_ref,
                 Av_ref, Aa_ref, Pv_ref, Pa_ref, Cv_ref, Ca_ref,
                 b1_ref, a1_ref, W2_ref, b2_ref, a2_ref, W3_ref, b3_ref,
                 Whcv_ref, Whca_ref, Whpv_ref, Whpa_ref, bh_ref, out_ref):
    hv = hv_ref[...]
    ha = ha_ref[...]
    cv = cv_ref[...]
    ca = ca_ref[...]

    M = BB * L
    hv2 = hv.reshape(M, VDIM)
    ha2 = ha.reshape(M, ADIM)
    hvp2 = (hv * cv[:, None, :]).reshape(M, VDIM)
    hap2 = (ha * ca[:, None, :]).reshape(M, ADIM)

    dot = functools.partial(jnp.dot, preferred_element_type=jnp.float32)
    h1 = (dot(hv2, Av_ref[...]) + dot(ha2, Aa_ref[...])
          + dot(hvp2, Pv_ref[...]) + dot(hap2, Pa_ref[...]))
    u = dot(cv, Cv_ref[...]) + dot(ca, Ca_ref[...]) + b1_ref[...]
    h1 = h1.reshape(BB, L, -1) + u[:, None, :]
    a1 = a1_ref[...]
    h1 = jnp.where(h1 > 0, h1, a1[None] * h1)

    h2 = dot(h1.reshape(M, -1), W2_ref[...]) + b2_ref[...]
    h2 = jnp.where(h2 > 0, h2, a2_ref[...] * h2)

    s = jnp.sum(h2.reshape(BB, L, -1) * W3_ref[...][None], axis=2) + b3_ref[0, 0]
    s = jnp.where(hvid_ref[...] != 0, s, -1e9)
    m = jnp.max(s, axis=1, keepdims=True)
    e = jnp.exp(s - m)
    w = e / jnp.sum(e, axis=1, keepdims=True)

    pooled_v = jnp.sum(hv * w[:, :, None], axis=1)
    pooled_a = jnp.sum(ha * w[:, :, None], axis=1)

    logit = (jnp.sum(cv * Whcv_ref[...], axis=1, keepdims=True)
             + jnp.sum(ca * Whca_ref[...], axis=1, keepdims=True)
             + jnp.sum(pooled_v * Whpv_ref[...], axis=1, keepdims=True)
             + jnp.sum(pooled_a * Whpa_ref[...], axis=1, keepdims=True)
             + bh_ref[0, 0])
    out_ref[...] = logit


def _tc_din(hv3, ha3, hvid, cv, ca, Av, Aa, Pv, Pa, Cv, Ca,
            b1, a1, W2, b2, a2, W3r, b3r, Whcv, Whca, Whpv, Whpa, bhr):
    grid = (B // BB,)
    full = lambda shp: pl.BlockSpec(shp, lambda i: (0,) * len(shp))
    return pl.pallas_call(
        _tc_din_body,
        grid=grid,
        in_specs=[
            pl.BlockSpec((BB, L, VDIM), lambda i: (i, 0, 0)),
            pl.BlockSpec((BB, L, ADIM), lambda i: (i, 0, 0)),
            pl.BlockSpec((BB, L), lambda i: (i, 0)),
            pl.BlockSpec((BB, VDIM), lambda i: (i, 0)),
            pl.BlockSpec((BB, ADIM), lambda i: (i, 0)),
            full((VDIM, 64)), full((ADIM, 64)),
            full((VDIM, 64)), full((ADIM, 64)),
            full((VDIM, 64)), full((ADIM, 64)),
            full((1, 64)), full((1, 64)),
            full((64, 32)), full((1, 32)), full((1, 32)),
            full((1, 32)), full((1, 1)),
            full((1, VDIM)), full((1, ADIM)),
            full((1, VDIM)), full((1, ADIM)), full((1, 1)),
        ],
        out_specs=pl.BlockSpec((BB, 1), lambda i: (i, 0)),
        out_shape=jax.ShapeDtypeStruct((B, 1), jnp.float32),
        compiler_params=pltpu.CompilerParams(
            dimension_semantics=("arbitrary",),
        ),
    )(hv3, ha3, hvid, cv, ca, Av, Aa, Pv, Pa, Cv, Ca,
      b1, a1, W2, b2, a2, W3r, b3r, Whcv, Whca, Whpv, Whpa, bhr)


def kernel(cand_video_id, cand_author_id, hist_video_id, hist_author_id,
           video_table, author_table,
           W1, b1, a1, W2, b2, a2, W3, b3, Wh, bh):
    hvid_flat = hist_video_id.reshape(NH)
    haid_flat = hist_author_id.reshape(NH)

    hv_flat, ha_flat, cv, ca = _sc_gather(
        hvid_flat, haid_flat, cand_video_id, cand_author_id,
        video_table, author_table)

    ITEM = VDIM + ADIM
    A = W1[0:ITEM] + W1[2 * ITEM:3 * ITEM]
    C = W1[ITEM:2 * ITEM] - W1[2 * ITEM:3 * ITEM]
    P = W1[3 * ITEM:4 * ITEM]

    out = _tc_din(
        hv_flat.reshape(B, L, VDIM), ha_flat.reshape(B, L, ADIM),
        hist_video_id, cv, ca,
        A[:VDIM], A[VDIM:], P[:VDIM], P[VDIM:], C[:VDIM], C[VDIM:],
        b1[None, :], a1[None, :], W2, b2[None, :], a2[None, :],
        W3[:, 0][None, :], b3[None, :],
        Wh[0:VDIM][None, :], Wh[VDIM:ITEM][None, :],
        Wh[ITEM:ITEM + VDIM][None, :], Wh[ITEM + VDIM:][None, :],
        bh[None, None])
    return out[:, 0]
